# baseline (speedup 1.0000x reference)
import functools

import jax
import jax.numpy as jnp
from jax import lax
from jax.experimental import pallas as pl
from jax.experimental.pallas import tpu as pltpu
from jax.experimental.pallas import tpu_sc as plsc

N_STOCKS = 2048
N_MARKET = 32
HIDDEN = 256
N_CLUSTERS = 8
N_SUBCLUSTERS = 4
N_HEADS = 4
DH = HIDDEN // N_HEADS
CSIZE = N_STOCKS // N_CLUSTERS
ISIZE = CSIZE // N_SUBCLUSTERS
QB = CSIZE
W3 = 3 * N_SUBCLUSTERS * HIDDEN
NEG = -1000000000.0
_HI = jax.lax.Precision.HIGHEST


def _norm_rows(v, eps=1e-12):
    n = jnp.sqrt(jnp.sum(v * v, axis=-1, keepdims=True))
    return v / jnp.maximum(n, eps)


def _rank_kernel(sims_ref, cl_ref, idxs_ref):
    sims_r = sims_ref[...]
    sims_c = jnp.transpose(sims_r)
    idx_c = jax.lax.broadcasted_iota(jnp.int32, (N_STOCKS, 1), 0)
    idx_r = jax.lax.broadcasted_iota(jnp.int32, (1, N_STOCKS), 1)

    lt = (sims_r < sims_c) | ((sims_r == sims_c) & (idx_r < idx_c))
    rank = jnp.sum(lt.astype(jnp.float32), axis=1, keepdims=True)
    cl_c = (N_CLUSTERS - 1
            - (rank * (1.0 / CSIZE)).astype(jnp.int32)).astype(jnp.int32)
    cl_r = jnp.transpose(cl_c)

    same = (cl_r == cl_c)
    pos = jnp.sum((same & (idx_r < idx_c)).astype(jnp.float32), axis=1,
                  keepdims=True)
    piota = jax.lax.broadcasted_iota(jnp.int32, (1, CSIZE), 1)
    pos_i = pos.astype(jnp.int32)
    jf_r = idx_r.astype(jnp.float32)

    cl_ref[...] = cl_r
    for c in range(N_CLUSTERS):
        memb_c = (cl_c == c)
        ph = ((pos_i == piota) & memb_c).astype(jnp.float32)
        row = jax.lax.dot_general(jf_r, ph, (((1,), (0,)), ((), ())),
                                  preferred_element_type=jnp.float32,
                                  precision=_HI)
        idxs_ref[c, :] = row.astype(jnp.int32)[0]


def _interval_sc_kernel(s_hbm, iv_hbm, s_v, iv_v):
    wid = lax.axis_index("s") * 2 + lax.axis_index("c")
    c = wid // N_SUBCLUSTERS
    q = wid % N_SUBCLUSTERS
    pltpu.sync_copy(s_hbm.at[c], s_v)
    for t in range(ISIZE // 16):
        base = q * ISIZE + t * 16
        s_mine = s_v[pl.ds(base, 16)]
        posv = lax.iota(jnp.int32, 16) + base

        def body(jb, cnt):
            sb = s_v[pl.ds(jb * 16, 16)]
            for l in range(16):
                j = jb * 16 + l
                sj = sb[l]
                beats = (sj < s_mine) | ((sj == s_mine) & (j < posv))
                cnt = cnt + jnp.where(beats, 1, 0)
            return cnt

        cnt = lax.fori_loop(0, CSIZE // 16, body,
                            jnp.zeros((16,), jnp.int32))
        iv_v[pl.ds(t * 16, 16)] = (N_SUBCLUSTERS - 1
                                   - jax.lax.shift_right_logical(cnt, 6))
    pltpu.sync_copy(iv_v, iv_hbm.at[c, pl.ds(q * ISIZE, ISIZE)])


_interval_sc = functools.partial(
    pl.kernel,
    mesh=plsc.VectorSubcoreMesh(core_axis_name="c", subcore_axis_name="s"),
    out_type=jax.ShapeDtypeStruct((N_CLUSTERS, CSIZE), jnp.int32),
    scratch_types=[
        pltpu.VMEM((CSIZE,), jnp.float32),
        pltpu.VMEM((ISIZE,), jnp.int32),
    ],
)(_interval_sc_kernel)


def _attn_kernel(x_ref, xs_ref, w_ref, b_ref, ivr_ref, ivc_ref,
                 wd_ref, bd_ref, g_ref, bln_ref, wc_ref, bc_ref, o_ref):
    xb = x_ref[...]
    xsum = xs_ref[...]
    iv_r = ivr_ref[0]
    iv_c = ivc_ref[0]

    acc = jnp.zeros((QB, HIDDEN), jnp.float32)
    for kidx in range(N_SUBCLUSTERS):
        off = 3 * kidx * HIDDEN
        wq = w_ref[:, off:off + HIDDEN]
        wk = w_ref[:, off + HIDDEN:off + 2 * HIDDEN]
        wv = w_ref[:, off + 2 * HIDDEN:off + 3 * HIDDEN]
        bq = b_ref[:, off:off + HIDDEN]
        bk = b_ref[:, off + HIDDEN:off + 2 * HIDDEN]
        bv = b_ref[:, off + 2 * HIDDEN:off + 3 * HIDDEN]
        q = jnp.dot(xb, wq, preferred_element_type=jnp.float32) + bq
        k = jnp.dot(xb, wk, preferred_element_type=jnp.float32) + bk
        v = jnp.dot(xb, wv, preferred_element_type=jnp.float32) + bv
        meanv = (jnp.dot(xsum, wv, preferred_element_type=jnp.float32)
                 + float(N_STOCKS) * bv) * (1.0 / N_STOCKS)

        member_q = (iv_c == kidx)
        member_k = (iv_r == kidx)
        addmask = jnp.where(member_q & member_k, 0.0, NEG)

        ctxs = []
        for h in range(N_HEADS):
            qh = q[:, h * DH:(h + 1) * DH]
            kh = k[:, h * DH:(h + 1) * DH]
            vh = v[:, h * DH:(h + 1) * DH]
            s = jax.lax.dot_general(qh, kh, (((1,), (1,)), ((), ())),
                                    preferred_element_type=jnp.float32)
            s = s * (1.0 / (DH ** 0.5)) + addmask
            m = jnp.max(s, axis=1, keepdims=True)
            p = jnp.exp(s - m)
            l = jnp.sum(p, axis=1, keepdims=True)
            ctxs.append(jnp.dot(p / l, vh,
                                preferred_element_type=jnp.float32))
        ctx = jnp.concatenate(ctxs, axis=1)
        ctx = jnp.where(member_q, ctx, meanv)

        out = jnp.dot(ctx, wd_ref[kidx],
                      preferred_element_type=jnp.float32) + bd_ref[kidx] + xb
        mu = jnp.mean(out, axis=1, keepdims=True)
        d = out - mu
        var = jnp.mean(d * d, axis=1, keepdims=True)
        y = d / jnp.sqrt(var + 1e-12) * g_ref[kidx] + bln_ref[kidx]

        acc = acc + jnp.dot(y, wc_ref[kidx],
                            preferred_element_type=jnp.float32)
    o_ref[...] = acc + bc_ref[...]


def kernel(stock_reps, market_reps, params):
    x = stock_reps
    mr = jnp.transpose(jnp.squeeze(market_reps, axis=0), (1, 0))
    sims = jnp.mean(_norm_rows(x) @ _norm_rows(mr).T, axis=1)

    cl_r, idxs = pl.pallas_call(
        _rank_kernel,
        out_shape=[
            jax.ShapeDtypeStruct((1, N_STOCKS), jnp.int32),
            jax.ShapeDtypeStruct((N_CLUSTERS, CSIZE), jnp.int32),
        ],
    )(sims.reshape(1, N_STOCKS))

    perm = idxs.reshape(N_STOCKS)
    x_perm = x[perm]

    s_rows = []
    for c in range(N_CLUSTERS):
        cs = x_perm[c * CSIZE:(c + 1) * CSIZE]
        centroid = jnp.mean(cs, axis=0, keepdims=True)
        s_rows.append(jnp.squeeze(_norm_rows(cs) @ _norm_rows(centroid).T))
    s8 = jnp.stack(s_rows)

    iv8 = _interval_sc(s8)

    attn = params['attn']
    wcat = jnp.concatenate(
        [attn[k][nm] for k in range(N_SUBCLUSTERS)
         for nm in ('Wq', 'Wk', 'Wv')], axis=1)
    bcat = jnp.concatenate(
        [attn[k][nm] for k in range(N_SUBCLUSTERS)
         for nm in ('bq', 'bk', 'bv')], axis=0)[None, :]

    xsum = jnp.sum(x_perm, axis=0, keepdims=True)

    wd = jnp.stack([attn[k]['Wd'] for k in range(N_SUBCLUSTERS)])
    bd = jnp.stack([attn[k]['bd'] for k in range(N_SUBCLUSTERS)])[:, None, :]
    g = jnp.stack([attn[k]['g'] for k in range(N_SUBCLUSTERS)])[:, None, :]
    bln = jnp.stack([attn[k]['b_ln']
                     for k in range(N_SUBCLUSTERS)])[:, None, :]
    wc = params['Wc'].reshape(N_SUBCLUSTERS, HIDDEN, HIDDEN)
    bc = params['bc'][None, :]
    iv_row = iv8.reshape(N_CLUSTERS, 1, CSIZE)
    iv_col = iv8.reshape(N_CLUSTERS, CSIZE, 1)

    out_perm = pl.pallas_call(
        _attn_kernel,
        grid=(N_CLUSTERS,),
        in_specs=[
            pl.BlockSpec((QB, HIDDEN), lambda i: (i, 0)),
            pl.BlockSpec((1, HIDDEN), lambda i: (0, 0)),
            pl.BlockSpec((HIDDEN, W3), lambda i: (0, 0)),
            pl.BlockSpec((1, W3), lambda i: (0, 0)),
            pl.BlockSpec((1, 1, CSIZE), lambda i: (i, 0, 0)),
            pl.BlockSpec((1, CSIZE, 1), lambda i: (i, 0, 0)),
            pl.BlockSpec((N_SUBCLUSTERS, HIDDEN, HIDDEN),
                         lambda i: (0, 0, 0)),
            pl.BlockSpec((N_SUBCLUSTERS, 1, HIDDEN), lambda i: (0, 0, 0)),
            pl.BlockSpec((N_SUBCLUSTERS, 1, HIDDEN), lambda i: (0, 0, 0)),
            pl.BlockSpec((N_SUBCLUSTERS, 1, HIDDEN), lambda i: (0, 0, 0)),
            pl.BlockSpec((N_SUBCLUSTERS, HIDDEN, HIDDEN),
                         lambda i: (0, 0, 0)),
            pl.BlockSpec((1, HIDDEN), lambda i: (0, 0)),
        ],
        out_specs=pl.BlockSpec((QB, HIDDEN), lambda i: (i, 0)),
        out_shape=jax.ShapeDtypeStruct((N_STOCKS, HIDDEN), jnp.float32),
    )(x_perm, xsum, wcat, bcat, iv_row, iv_col, wd, bd, g, bln, wc, bc)

    reps = jnp.zeros((N_STOCKS, HIDDEN), jnp.float32).at[perm].set(out_perm)
    return reps, cl_r.reshape(-1), sims


# speedup vs baseline: 1.5697x; 1.5697x over previous
import functools

import jax
import jax.numpy as jnp
from jax import lax
from jax.experimental import pallas as pl
from jax.experimental.pallas import tpu as pltpu
from jax.experimental.pallas import tpu_sc as plsc

N_STOCKS = 2048
N_MARKET = 32
HIDDEN = 256
N_CLUSTERS = 8
N_SUBCLUSTERS = 4
N_HEADS = 4
DH = HIDDEN // N_HEADS
CSIZE = N_STOCKS // N_CLUSTERS
ISIZE = CSIZE // N_SUBCLUSTERS
QB = CSIZE
W3 = 3 * N_SUBCLUSTERS * HIDDEN
NEG = -1000000000.0
_HI = jax.lax.Precision.HIGHEST


def _norm_rows(v, eps=1e-12):
    n = jnp.sqrt(jnp.sum(v * v, axis=-1, keepdims=True))
    return v / jnp.maximum(n, eps)


def _rank_kernel(sims_ref, cl_ref, idxs_ref):
    sims_r = sims_ref[...]
    sims_c = jnp.transpose(sims_r)
    idx_c = jax.lax.broadcasted_iota(jnp.int32, (N_STOCKS, 1), 0)
    idx_r = jax.lax.broadcasted_iota(jnp.int32, (1, N_STOCKS), 1)

    lt = (sims_r < sims_c) | ((sims_r == sims_c) & (idx_r < idx_c))
    rank = jnp.sum(lt.astype(jnp.float32), axis=1, keepdims=True)
    cl_c = (N_CLUSTERS - 1
            - (rank * (1.0 / CSIZE)).astype(jnp.int32)).astype(jnp.int32)
    cl_r = jnp.transpose(cl_c)

    same = (cl_r == cl_c)
    pos = jnp.sum((same & (idx_r < idx_c)).astype(jnp.float32), axis=1,
                  keepdims=True)
    piota = jax.lax.broadcasted_iota(jnp.int32, (1, CSIZE), 1)
    pos_i = pos.astype(jnp.int32)
    jf_r = idx_r.astype(jnp.float32)

    cl_ref[...] = cl_r
    for c in range(N_CLUSTERS):
        memb_c = (cl_c == c)
        ph = ((pos_i == piota) & memb_c).astype(jnp.float32)
        row = jax.lax.dot_general(jf_r, ph, (((1,), (0,)), ((), ())),
                                  preferred_element_type=jnp.float32,
                                  precision=_HI)
        idxs_ref[c, :] = row.astype(jnp.int32)[0]


def _interval_sc_kernel(s_hbm, iv_hbm, s_v, iv_v):
    wid = lax.axis_index("s") * 2 + lax.axis_index("c")
    c = wid // N_SUBCLUSTERS
    q = wid % N_SUBCLUSTERS
    pltpu.sync_copy(s_hbm.at[c], s_v)
    for t in range(ISIZE // 16):
        base = q * ISIZE + t * 16
        s_mine = s_v[pl.ds(base, 16)]
        posv = lax.iota(jnp.int32, 16) + base

        def body(jb, cnt):
            sb = s_v[pl.ds(jb * 16, 16)]
            for l in range(16):
                j = jb * 16 + l
                sj = sb[l]
                beats = (sj < s_mine) | ((sj == s_mine) & (j < posv))
                cnt = cnt + jnp.where(beats, 1, 0)
            return cnt

        cnt = lax.fori_loop(0, CSIZE // 16, body,
                            jnp.zeros((16,), jnp.int32))
        iv_v[pl.ds(t * 16, 16)] = (N_SUBCLUSTERS - 1
                                   - jax.lax.shift_right_logical(cnt, 6))
    pltpu.sync_copy(iv_v, iv_hbm.at[c, pl.ds(q * ISIZE, ISIZE)])


_interval_sc = functools.partial(
    pl.kernel,
    mesh=plsc.VectorSubcoreMesh(core_axis_name="c", subcore_axis_name="s"),
    out_type=jax.ShapeDtypeStruct((N_CLUSTERS, CSIZE), jnp.int32),
    scratch_types=[
        pltpu.VMEM((CSIZE,), jnp.float32),
        pltpu.VMEM((ISIZE,), jnp.int32),
    ],
)(_interval_sc_kernel)


def _attn_kernel(x_ref, xs_ref, w_ref, b_ref, ivr_ref, ivc_ref,
                 wd_ref, bd_ref, g_ref, bln_ref, wc_ref, bc_ref, o_ref):
    xb = x_ref[...]
    xsum = xs_ref[...]
    iv_r = ivr_ref[0]
    iv_c = ivc_ref[0]

    qs, ks, vs, mvs = [], [], [], []
    for kidx in range(N_SUBCLUSTERS):
        off = 3 * kidx * HIDDEN
        wq = w_ref[:, off:off + HIDDEN]
        wk = w_ref[:, off + HIDDEN:off + 2 * HIDDEN]
        wv = w_ref[:, off + 2 * HIDDEN:off + 3 * HIDDEN]
        bq = b_ref[:, off:off + HIDDEN]
        bk = b_ref[:, off + HIDDEN:off + 2 * HIDDEN]
        bv = b_ref[:, off + 2 * HIDDEN:off + 3 * HIDDEN]
        qs.append(jnp.dot(xb, wq, preferred_element_type=jnp.float32) + bq)
        ks.append(jnp.dot(xb, wk, preferred_element_type=jnp.float32) + bk)
        vs.append(jnp.dot(xb, wv, preferred_element_type=jnp.float32) + bv)
        mvs.append((jnp.dot(xsum, wv, preferred_element_type=jnp.float32)
                    + float(N_STOCKS) * bv) * (1.0 / N_STOCKS))

    qm = qs[0]
    km = ks[0]
    vm = vs[0]
    for kidx in range(1, N_SUBCLUSTERS):
        sel = (iv_c == kidx)
        qm = jnp.where(sel, qs[kidx], qm)
        km = jnp.where(sel, ks[kidx], km)
        vm = jnp.where(sel, vs[kidx], vm)

    same_iv = (iv_c == iv_r)
    addmask = jnp.where(same_iv, 0.0, NEG)

    ctxs = []
    for h in range(N_HEADS):
        qh = qm[:, h * DH:(h + 1) * DH]
        kh = km[:, h * DH:(h + 1) * DH]
        vh = vm[:, h * DH:(h + 1) * DH]
        s = jax.lax.dot_general(qh, kh, (((1,), (1,)), ((), ())),
                                preferred_element_type=jnp.float32)
        s = s * (1.0 / (DH ** 0.5)) + addmask
        m = jnp.max(s, axis=1, keepdims=True)
        p = jnp.exp(s - m)
        l = jnp.sum(p, axis=1, keepdims=True)
        ctxs.append(jnp.dot(p / l, vh, preferred_element_type=jnp.float32))
    ctx_m = jnp.concatenate(ctxs, axis=1)

    acc = jnp.zeros((QB, HIDDEN), jnp.float32)
    for kidx in range(N_SUBCLUSTERS):
        ctx = jnp.where(iv_c == kidx, ctx_m, mvs[kidx])
        out = jnp.dot(ctx, wd_ref[kidx],
                      preferred_element_type=jnp.float32) + bd_ref[kidx] + xb
        mu = jnp.mean(out, axis=1, keepdims=True)
        d = out - mu
        var = jnp.mean(d * d, axis=1, keepdims=True)
        y = d / jnp.sqrt(var + 1e-12) * g_ref[kidx] + bln_ref[kidx]
        acc = acc + jnp.dot(y, wc_ref[kidx],
                            preferred_element_type=jnp.float32)
    o_ref[...] = acc + bc_ref[...]


def kernel(stock_reps, market_reps, params):
    x = stock_reps
    mr = jnp.transpose(jnp.squeeze(market_reps, axis=0), (1, 0))
    sims = jnp.mean(_norm_rows(x) @ _norm_rows(mr).T, axis=1)

    cl_r, idxs = pl.pallas_call(
        _rank_kernel,
        out_shape=[
            jax.ShapeDtypeStruct((1, N_STOCKS), jnp.int32),
            jax.ShapeDtypeStruct((N_CLUSTERS, CSIZE), jnp.int32),
        ],
    )(sims.reshape(1, N_STOCKS))

    perm = idxs.reshape(N_STOCKS)
    x_perm = x[perm]

    s_rows = []
    for c in range(N_CLUSTERS):
        cs = x_perm[c * CSIZE:(c + 1) * CSIZE]
        centroid = jnp.mean(cs, axis=0, keepdims=True)
        s_rows.append(jnp.squeeze(_norm_rows(cs) @ _norm_rows(centroid).T))
    s8 = jnp.stack(s_rows)

    iv8 = _interval_sc(s8)

    attn = params['attn']
    wcat = jnp.concatenate(
        [attn[k][nm] for k in range(N_SUBCLUSTERS)
         for nm in ('Wq', 'Wk', 'Wv')], axis=1)
    bcat = jnp.concatenate(
        [attn[k][nm] for k in range(N_SUBCLUSTERS)
         for nm in ('bq', 'bk', 'bv')], axis=0)[None, :]

    xsum = jnp.sum(x_perm, axis=0, keepdims=True)

    wd = jnp.stack([attn[k]['Wd'] for k in range(N_SUBCLUSTERS)])
    bd = jnp.stack([attn[k]['bd'] for k in range(N_SUBCLUSTERS)])[:, None, :]
    g = jnp.stack([attn[k]['g'] for k in range(N_SUBCLUSTERS)])[:, None, :]
    bln = jnp.stack([attn[k]['b_ln']
                     for k in range(N_SUBCLUSTERS)])[:, None, :]
    wc = params['Wc'].reshape(N_SUBCLUSTERS, HIDDEN, HIDDEN)
    bc = params['bc'][None, :]
    iv_row = iv8.reshape(N_CLUSTERS, 1, CSIZE)
    iv_col = iv8.reshape(N_CLUSTERS, CSIZE, 1)

    out_perm = pl.pallas_call(
        _attn_kernel,
        grid=(N_CLUSTERS,),
        in_specs=[
            pl.BlockSpec((QB, HIDDEN), lambda i: (i, 0)),
            pl.BlockSpec((1, HIDDEN), lambda i: (0, 0)),
            pl.BlockSpec((HIDDEN, W3), lambda i: (0, 0)),
            pl.BlockSpec((1, W3), lambda i: (0, 0)),
            pl.BlockSpec((1, 1, CSIZE), lambda i: (i, 0, 0)),
            pl.BlockSpec((1, CSIZE, 1), lambda i: (i, 0, 0)),
            pl.BlockSpec((N_SUBCLUSTERS, HIDDEN, HIDDEN),
                         lambda i: (0, 0, 0)),
            pl.BlockSpec((N_SUBCLUSTERS, 1, HIDDEN), lambda i: (0, 0, 0)),
            pl.BlockSpec((N_SUBCLUSTERS, 1, HIDDEN), lambda i: (0, 0, 0)),
            pl.BlockSpec((N_SUBCLUSTERS, 1, HIDDEN), lambda i: (0, 0, 0)),
            pl.BlockSpec((N_SUBCLUSTERS, HIDDEN, HIDDEN),
                         lambda i: (0, 0, 0)),
            pl.BlockSpec((1, HIDDEN), lambda i: (0, 0)),
        ],
        out_specs=pl.BlockSpec((QB, HIDDEN), lambda i: (i, 0)),
        out_shape=jax.ShapeDtypeStruct((N_STOCKS, HIDDEN), jnp.float32),
    )(x_perm, xsum, wcat, bcat, iv_row, iv_col, wd, bd, g, bln, wc, bc)

    reps = jnp.zeros((N_STOCKS, HIDDEN), jnp.float32).at[perm].set(out_perm)
    return reps, cl_r.reshape(-1), sims
